# baseline (speedup 1.0000x reference)
import jax
import jax.numpy as jnp
from jax import lax
from jax.experimental import pallas as pl
from jax.experimental.pallas import tpu as pltpu
from jax.experimental.pallas import tpu_sc as plsc

CODEBOOK_SIZE = 1024
CODEBOOK_DIM = 64
PAD_DIM = 128
COMMITMENT_COST = 0.25

TILE_N = 512

NUM_WORKERS = 32
IDX_CHUNK = 128
NBANK = 2
NGROUP = 3


def _argmin_kernel(x_ref, c_ref, idx_ref, loss_ref, acc_ref):
    i = pl.program_id(0)
    nsteps = pl.num_programs(0)

    x = x_ref[...]
    c = c_ref[...]

    x2 = jnp.sum(x * x, axis=1, keepdims=True)
    c2 = jnp.sum(c * c, axis=1)
    xc = jax.lax.dot_general(
        x, c, (((1,), (1,)), ((), ())),
        preferred_element_type=jnp.float32)
    d = x2 + c2[None, :] - 2.0 * xc

    dmin = jnp.min(d, axis=1, keepdims=True)
    iota = jax.lax.broadcasted_iota(jnp.int32, d.shape, 1)
    idx = jnp.min(jnp.where(d == dmin, iota, CODEBOOK_SIZE), axis=1)
    idx_ref[...] = idx.astype(jnp.int32)

    part = jnp.sum(dmin)

    @pl.when(i == 0)
    def _():
        acc_ref[0, 0] = part

    @pl.when(i != 0)
    def _():
        acc_ref[0, 0] = acc_ref[0, 0] + part

    @pl.when(i == nsteps - 1)
    def _():
        total_elems = nsteps * TILE_N * CODEBOOK_DIM
        loss_ref[0, 0] = acc_ref[0, 0] * (COMMITMENT_COST / total_elems)


def _make_sc_gather(n_rows):
    rows_per_w = n_rows // NUM_WORKERS
    n_chunks = rows_per_w // IDX_CHUNK
    mesh = plsc.VectorSubcoreMesh(core_axis_name="c", subcore_axis_name="s")

    def sc_gather(codebook_padded, idx_flat):
        @pl.kernel(
            mesh=mesh,
            out_type=jax.ShapeDtypeStruct((n_rows, PAD_DIM), jnp.float32),
            scratch_types=[
                pltpu.VMEM((rows_per_w,), jnp.int32),
                pltpu.VMEM((NBANK, NGROUP, IDX_CHUNK, PAD_DIM), jnp.float32),
                pltpu.SemaphoreType.DMA((NBANK,)),
                pltpu.SemaphoreType.DMA((NBANK,)),
            ],
        )
        def body(cb_hbm, idx_hbm, out_hbm, idx_v, rows_v, gsem, wsem):
            wid = lax.axis_index("s") * 2 + lax.axis_index("c")
            base = wid * rows_per_w
            pltpu.sync_copy(idx_hbm.at[pl.ds(base, rows_per_w)], idx_v)

            def fire(j):
                bank = (j // NGROUP) % NBANK
                slot = j % NGROUP
                return pltpu.async_copy(
                    cb_hbm.at[idx_v.at[pl.ds(j * IDX_CHUNK, IDX_CHUNK)]],
                    rows_v.at[bank, slot],
                    gsem.at[bank])

            def write(j):
                bank = (j // NGROUP) % NBANK
                slot = j % NGROUP
                return pltpu.async_copy(
                    rows_v.at[bank, slot],
                    out_hbm.at[pl.ds(base + j * IDX_CHUNK, IDX_CHUNK)],
                    wsem.at[bank])

            n_groups = n_chunks // NGROUP
            gathers, writes = {}, {}
            for g in range(n_groups):
                for j in range(g * NGROUP, (g + 1) * NGROUP):
                    if g >= NBANK:
                        writes[j - NBANK * NGROUP].wait()
                    gathers[j] = fire(j)
                for j in range(g * NGROUP, (g + 1) * NGROUP):
                    gathers[j].wait()
                    writes[j] = write(j)
            for g in range(max(n_groups - NBANK, 0), n_groups):
                for j in range(g * NGROUP, (g + 1) * NGROUP):
                    writes[j].wait()

        return body(codebook_padded, idx_flat)

    return sc_gather


def kernel(inputs, codebook):
    batch, time_steps, dim = inputs.shape
    n = batch * time_steps
    flat = inputs.reshape(n, dim)
    grid = n // TILE_N

    idx, loss = pl.pallas_call(
        _argmin_kernel,
        grid=(grid,),
        in_specs=[
            pl.BlockSpec((TILE_N, dim), lambda i: (i, 0)),
            pl.BlockSpec((CODEBOOK_SIZE, dim), lambda i: (0, 0)),
        ],
        out_specs=[
            pl.BlockSpec((TILE_N,), lambda i: (i,)),
            pl.BlockSpec(memory_space=pltpu.SMEM),
        ],
        out_shape=[
            jax.ShapeDtypeStruct((n,), jnp.int32),
            jax.ShapeDtypeStruct((1, 1), jnp.float32),
        ],
        scratch_shapes=[pltpu.SMEM((1, 1), jnp.float32)],
    )(flat, codebook)

    cb_padded = jnp.pad(codebook, ((0, 0), (0, PAD_DIM - dim)))
    q_padded = _make_sc_gather(n)(cb_padded, idx)
    q = q_padded[:, :dim]

    quantized = q.reshape(batch, time_steps, dim)
    indices = idx.reshape(batch, time_steps)
    return quantized, indices, loss[0, 0]


# speedup vs baseline: 1.0429x; 1.0429x over previous
import jax
import jax.numpy as jnp
from jax import lax
from jax.experimental import pallas as pl
from jax.experimental.pallas import tpu as pltpu
from jax.experimental.pallas import tpu_sc as plsc

CODEBOOK_SIZE = 1024
CODEBOOK_DIM = 64
PAD_DIM = 128
COMMITMENT_COST = 0.25

TILE_N = 512

NUM_WORKERS = 32
IDX_CHUNK = 128
NBANK = 2
NGROUP = 3


def _argmin_kernel(x_ref, c_ref, idx_ref, loss_ref, acc_ref):
    i = pl.program_id(0)
    nsteps = pl.num_programs(0)

    x = x_ref[...]
    c = c_ref[...]

    x2 = jnp.sum(x * x, axis=1, keepdims=True)
    c2 = jnp.sum(c * c, axis=1)
    xc2 = jax.lax.dot_general(
        -2.0 * x, c, (((1,), (1,)), ((), ())),
        preferred_element_type=jnp.float32)
    d = (x2 + c2[None, :]) + xc2

    dmin = jnp.min(d, axis=1, keepdims=True)
    idx = jnp.argmin(d, axis=1)
    idx_ref[...] = idx.astype(jnp.int32)

    part = jnp.sum(dmin)

    @pl.when(i == 0)
    def _():
        acc_ref[0, 0] = part

    @pl.when(i != 0)
    def _():
        acc_ref[0, 0] = acc_ref[0, 0] + part

    @pl.when(i == nsteps - 1)
    def _():
        total_elems = nsteps * TILE_N * CODEBOOK_DIM
        loss_ref[0, 0] = acc_ref[0, 0] * (COMMITMENT_COST / total_elems)


def _make_sc_gather(n_rows):
    rows_per_w = n_rows // NUM_WORKERS
    n_chunks = rows_per_w // IDX_CHUNK
    mesh = plsc.VectorSubcoreMesh(core_axis_name="c", subcore_axis_name="s")

    def sc_gather(codebook_padded, idx_flat):
        @pl.kernel(
            mesh=mesh,
            out_type=jax.ShapeDtypeStruct((n_rows, PAD_DIM), jnp.float32),
            scratch_types=[
                pltpu.VMEM((rows_per_w,), jnp.int32),
                pltpu.VMEM((NBANK, NGROUP, IDX_CHUNK, PAD_DIM), jnp.float32),
                pltpu.SemaphoreType.DMA((NBANK,)),
                pltpu.SemaphoreType.DMA((NBANK,)),
            ],
        )
        def body(cb_hbm, idx_hbm, out_hbm, idx_v, rows_v, gsem, wsem):
            wid = lax.axis_index("s") * 2 + lax.axis_index("c")
            base = wid * rows_per_w
            pltpu.sync_copy(idx_hbm.at[pl.ds(base, rows_per_w)], idx_v)

            def fire(j):
                bank = (j // NGROUP) % NBANK
                slot = j % NGROUP
                return pltpu.async_copy(
                    cb_hbm.at[idx_v.at[pl.ds(j * IDX_CHUNK, IDX_CHUNK)]],
                    rows_v.at[bank, slot],
                    gsem.at[bank])

            def write(j):
                bank = (j // NGROUP) % NBANK
                slot = j % NGROUP
                return pltpu.async_copy(
                    rows_v.at[bank, slot],
                    out_hbm.at[pl.ds(base + j * IDX_CHUNK, IDX_CHUNK)],
                    wsem.at[bank])

            n_groups = n_chunks // NGROUP
            gathers, writes = {}, {}
            for g in range(n_groups):
                for j in range(g * NGROUP, (g + 1) * NGROUP):
                    if g >= NBANK:
                        writes[j - NBANK * NGROUP].wait()
                    gathers[j] = fire(j)
                for j in range(g * NGROUP, (g + 1) * NGROUP):
                    gathers[j].wait()
                    writes[j] = write(j)
            for g in range(max(n_groups - NBANK, 0), n_groups):
                for j in range(g * NGROUP, (g + 1) * NGROUP):
                    writes[j].wait()

        return body(codebook_padded, idx_flat)

    return sc_gather


def kernel(inputs, codebook):
    batch, time_steps, dim = inputs.shape
    n = batch * time_steps
    flat = inputs.reshape(n, dim)
    grid = n // TILE_N

    idx, loss = pl.pallas_call(
        _argmin_kernel,
        grid=(grid,),
        in_specs=[
            pl.BlockSpec((TILE_N, dim), lambda i: (i, 0)),
            pl.BlockSpec((CODEBOOK_SIZE, dim), lambda i: (0, 0)),
        ],
        out_specs=[
            pl.BlockSpec((TILE_N,), lambda i: (i,)),
            pl.BlockSpec(memory_space=pltpu.SMEM),
        ],
        out_shape=[
            jax.ShapeDtypeStruct((n,), jnp.int32),
            jax.ShapeDtypeStruct((1, 1), jnp.float32),
        ],
        scratch_shapes=[pltpu.SMEM((1, 1), jnp.float32)],
    )(flat, codebook)

    cb_padded = jnp.pad(codebook, ((0, 0), (0, PAD_DIM - dim)))
    q_padded = _make_sc_gather(n)(cb_padded, idx)
    q = q_padded[:, :dim]

    quantized = q.reshape(batch, time_steps, dim)
    indices = idx.reshape(batch, time_steps)
    return quantized, indices, loss[0, 0]
